# baseline (speedup 1.0000x reference)
import jax
import jax.numpy as jnp
from jax import lax
from jax.experimental import pallas as pl
from jax.experimental.pallas import tpu as pltpu

D_MODEL_K = 2048
N_EXP = 16
BLK = 512


def _router_body(x_ref, w_ref, b_ref, wout_ref, iout_ref, cv_ref, esum_ref):
    i = pl.program_id(0)
    nblk = pl.num_programs(0)

    @pl.when(i == 0)
    def _init():
        esum_ref[...] = jnp.zeros_like(esum_ref)

    logits = jnp.dot(x_ref[...], w_ref[...], preferred_element_type=jnp.float32)
    logits = logits + b_ref[...]
    m = jnp.max(logits, axis=-1, keepdims=True)
    e = jnp.exp(logits - m)
    s = jnp.sum(e, axis=-1, keepdims=True)
    p = e / s

    esum_ref[...] += jnp.sum(p, axis=0, keepdims=True)

    iota = lax.broadcasted_iota(jnp.int32, (BLK, N_EXP), 1)
    m1 = jnp.max(p, axis=-1, keepdims=True)
    i1 = jnp.min(jnp.where(p == m1, iota, N_EXP), axis=-1, keepdims=True)
    p2 = jnp.where(iota == i1, -1.0, p)
    m2 = jnp.max(p2, axis=-1, keepdims=True)
    i2 = jnp.min(jnp.where(p2 == m2, iota, N_EXP), axis=-1, keepdims=True)

    tot = m1 + m2
    wout_ref[...] = jnp.concatenate([m1 / tot, m2 / tot], axis=1)
    iout_ref[...] = jnp.concatenate([i1, i2], axis=1)

    @pl.when(i == nblk - 1)
    def _finish():
        sums = esum_ref[...]
        f = sums / jnp.sum(sums)
        mean = jnp.sum(f) / N_EXP
        var = jnp.sum((f - mean) ** 2) / N_EXP
        cv_ref[...] = jnp.sqrt(var).reshape(1, 1) / mean


def kernel(x, W, b):
    B, T, d = x.shape
    n = B * T
    x_flat = x.reshape(n, d)
    b2 = b.reshape(1, N_EXP)
    nblk = n // BLK

    wout, iout, cv = pl.pallas_call(
        _router_body,
        grid=(nblk,),
        in_specs=[
            pl.BlockSpec((BLK, d), lambda i: (i, 0)),
            pl.BlockSpec((d, N_EXP), lambda i: (0, 0)),
            pl.BlockSpec((1, N_EXP), lambda i: (0, 0)),
        ],
        out_specs=[
            pl.BlockSpec((BLK, 2), lambda i: (i, 0)),
            pl.BlockSpec((BLK, 2), lambda i: (i, 0)),
            pl.BlockSpec((1, 1), lambda i: (0, 0)),
        ],
        out_shape=[
            jax.ShapeDtypeStruct((n, 2), jnp.float32),
            jax.ShapeDtypeStruct((n, 2), jnp.int32),
            jax.ShapeDtypeStruct((1, 1), jnp.float32),
        ],
        scratch_shapes=[pltpu.VMEM((1, N_EXP), jnp.float32)],
    )(x_flat, W, b2)

    return (wout.reshape(B, T, 2), iout.reshape(B, T, 2), cv.reshape(()))


# speedup vs baseline: 1.2201x; 1.2201x over previous
import jax
import jax.numpy as jnp
from jax import lax
from jax.experimental import pallas as pl
from jax.experimental.pallas import tpu as pltpu

D_MODEL_K = 2048
N_EXP = 16
BLK = 2048


def _router_body(x_ref, w_ref, b_ref, wout_ref, iout_ref, cv_ref, esum_ref):
    i = pl.program_id(0)
    nblk = pl.num_programs(0)

    @pl.when(i == 0)
    def _init():
        esum_ref[...] = jnp.zeros_like(esum_ref)

    logits = jnp.dot(x_ref[...], w_ref[...], preferred_element_type=jnp.float32)
    logits = logits + b_ref[...]
    m = jnp.max(logits, axis=-1, keepdims=True)
    e = jnp.exp(logits - m)
    s = jnp.sum(e, axis=-1, keepdims=True)
    p = e / s

    esum_ref[...] += jnp.sum(p, axis=0, keepdims=True)

    iota = lax.broadcasted_iota(jnp.int32, (BLK, N_EXP), 1)
    m1 = jnp.max(p, axis=-1, keepdims=True)
    i1 = jnp.min(jnp.where(p == m1, iota, N_EXP), axis=-1, keepdims=True)
    p2 = jnp.where(iota == i1, -1.0, p)
    m2 = jnp.max(p2, axis=-1, keepdims=True)
    i2 = jnp.min(jnp.where(p2 == m2, iota, N_EXP), axis=-1, keepdims=True)

    tot = m1 + m2
    wout_ref[...] = jnp.concatenate([m1 / tot, m2 / tot], axis=1)
    iout_ref[...] = jnp.concatenate([i1, i2], axis=1)

    @pl.when(i == nblk - 1)
    def _finish():
        sums = esum_ref[...]
        f = sums / jnp.sum(sums)
        mean = jnp.sum(f) / N_EXP
        var = jnp.sum((f - mean) ** 2) / N_EXP
        cv_ref[...] = jnp.sqrt(var).reshape(1, 1) / mean


def kernel(x, W, b):
    B, T, d = x.shape
    n = B * T
    x_flat = x.reshape(n, d)
    b2 = b.reshape(1, N_EXP)
    nblk = n // BLK

    wout, iout, cv = pl.pallas_call(
        _router_body,
        grid=(nblk,),
        in_specs=[
            pl.BlockSpec((BLK, d), lambda i: (i, 0)),
            pl.BlockSpec((d, N_EXP), lambda i: (0, 0)),
            pl.BlockSpec((1, N_EXP), lambda i: (0, 0)),
        ],
        out_specs=[
            pl.BlockSpec((BLK, 2), lambda i: (i, 0)),
            pl.BlockSpec((BLK, 2), lambda i: (i, 0)),
            pl.BlockSpec((1, 1), lambda i: (0, 0)),
        ],
        out_shape=[
            jax.ShapeDtypeStruct((n, 2), jnp.float32),
            jax.ShapeDtypeStruct((n, 2), jnp.int32),
            jax.ShapeDtypeStruct((1, 1), jnp.float32),
        ],
        scratch_shapes=[pltpu.VMEM((1, N_EXP), jnp.float32)],
    )(x_flat, W, b2)

    return (wout.reshape(B, T, 2), iout.reshape(B, T, 2), cv.reshape(()))


# speedup vs baseline: 1.2226x; 1.0021x over previous
import jax
import jax.numpy as jnp
from jax import lax
from jax.experimental import pallas as pl
from jax.experimental.pallas import tpu as pltpu

N_EXP = 16
BLK = 1024
NWIN = 2


def _route_block(x, w, b, wout_ref, iout_ref, esum_ref, row0):
    logits = jnp.dot(x, w, preferred_element_type=jnp.float32) + b
    m = jnp.max(logits, axis=-1, keepdims=True)
    e = jnp.exp(logits - m)
    s = jnp.sum(e, axis=-1, keepdims=True)
    p = e / s

    esum_ref[...] += jnp.sum(p, axis=0, keepdims=True)

    iota = lax.broadcasted_iota(jnp.int32, (BLK, N_EXP), 1)
    m1 = jnp.max(p, axis=-1, keepdims=True)
    i1 = jnp.min(jnp.where(p == m1, iota, N_EXP), axis=-1, keepdims=True)
    p2 = jnp.where(iota == i1, -1.0, p)
    m2 = jnp.max(p2, axis=-1, keepdims=True)
    i2 = jnp.min(jnp.where(p2 == m2, iota, N_EXP), axis=-1, keepdims=True)

    tot = m1 + m2
    wout_ref[pl.ds(row0, BLK), :] = jnp.concatenate([m1 / tot, m2 / tot], axis=1)
    iout_ref[pl.ds(row0, BLK), :] = jnp.concatenate([i1, i2], axis=1)


def _router_body(xa_ref, xb_ref, w_ref, b_ref, wout_ref, iout_ref, cv_ref,
                 esum_ref):
    i = pl.program_id(0)
    nblk = pl.num_programs(0)

    @pl.when(i == 0)
    def _init():
        esum_ref[...] = jnp.zeros_like(esum_ref)

    w = w_ref[...]
    b = b_ref[...]
    _route_block(xa_ref[...], w, b, wout_ref, iout_ref, esum_ref, 0)
    _route_block(xb_ref[...], w, b, wout_ref, iout_ref, esum_ref, BLK)

    @pl.when(i == nblk - 1)
    def _finish():
        sums = esum_ref[...]
        f = sums / jnp.sum(sums)
        mean = jnp.sum(f) / N_EXP
        var = jnp.sum((f - mean) ** 2) / N_EXP
        cv_ref[...] = jnp.sqrt(var).reshape(1, 1) / mean


def kernel(x, W, b):
    B, T, d = x.shape
    n = B * T
    x_flat = x.reshape(n, d)
    b2 = b.reshape(1, N_EXP)
    nblk = n // (BLK * NWIN)

    wout, iout, cv = pl.pallas_call(
        _router_body,
        grid=(nblk,),
        in_specs=[
            pl.BlockSpec((BLK, d), lambda i: (2 * i, 0)),
            pl.BlockSpec((BLK, d), lambda i: (2 * i + 1, 0)),
            pl.BlockSpec((d, N_EXP), lambda i: (0, 0)),
            pl.BlockSpec((1, N_EXP), lambda i: (0, 0)),
        ],
        out_specs=[
            pl.BlockSpec((BLK * NWIN, 2), lambda i: (i, 0)),
            pl.BlockSpec((BLK * NWIN, 2), lambda i: (i, 0)),
            pl.BlockSpec((1, 1), lambda i: (0, 0)),
        ],
        out_shape=[
            jax.ShapeDtypeStruct((n, 2), jnp.float32),
            jax.ShapeDtypeStruct((n, 2), jnp.int32),
            jax.ShapeDtypeStruct((1, 1), jnp.float32),
        ],
        scratch_shapes=[pltpu.VMEM((1, N_EXP), jnp.float32)],
    )(x_flat, x_flat, W, b2)

    return (wout.reshape(B, T, 2), iout.reshape(B, T, 2), cv.reshape(()))
